# Initial kernel scaffold; baseline (speedup 1.0000x reference)
import functools

import jax
import jax.numpy as jnp
from jax import lax
from jax.experimental import pallas as pl
from jax.experimental.pallas import tpu as pltpu
from jax.experimental.pallas import tpu_sc as plsc

MPE_ = 128
DIM_ = 64
B_ = 16
L_ = 200
LPAD_ = 224
NW_ = 32


def _avg_body(b_ref, ax_ref, ay_ref):
    ax_ref[...] = (b_ref[0] * float(MPE_) + b_ref[2] * float(MPE_)) * 0.5
    ay_ref[...] = (b_ref[1] * float(MPE_) + b_ref[3] * float(MPE_)) * 0.5


_avg_call = pl.pallas_call(
    _avg_body,
    out_shape=(
        jax.ShapeDtypeStruct((B_, LPAD_), jnp.float32),
        jax.ShapeDtypeStruct((B_, LPAD_), jnp.float32),
    ),
)


def _tab_body(x_ref, y_ref, o_ref):
    o_ref[...] = x_ref[...][:, None, :] + y_ref[...][None, :, :]


_tab_call = pl.pallas_call(
    _tab_body,
    grid=(8,),
    in_specs=[
        pl.BlockSpec((32, DIM_), lambda i: (i, 0)),
        pl.BlockSpec((256, DIM_), lambda i: (0, 0)),
    ],
    out_specs=pl.BlockSpec((32, 256, DIM_), lambda i: (i, 0, 0)),
    out_shape=jax.ShapeDtypeStruct((256, 256, DIM_), jnp.float32),
)


def _sc_body(tab_hbm, ax_hbm, ay_hbm, out_hbm, axv, ayv, idxv, rows, sem):
    cid = lax.axis_index("c")
    sid = lax.axis_index("s")
    b = sid
    i0 = cid * (L_ // 2)

    pltpu.sync_copy(ax_hbm.at[b], axv)
    pltpu.sync_copy(ay_hbm.at[b], ayv)

    def unit(i_rel, carry):
        i = i0 + i_rel
        axi = axv[i]
        ayi = ayv[i]
        for k in range(LPAD_ // 16):
            aj = axv[pl.ds(k * 16, 16)]
            dx = axi - aj + float(MPE_)
            ix = jnp.clip(dx, 0.0, float(2 * MPE_ - 1)).astype(jnp.int32)
            bj = ayv[pl.ds(k * 16, 16)]
            dy = ayi - bj + float(MPE_)
            iy = jnp.clip(dy, 0.0, float(2 * MPE_ - 1)).astype(jnp.int32)
            idx16 = ix * 256 + iy
            idxv[k // 7, pl.ds((k % 7) * 16, 16)] = idx16
        cp0 = pltpu.async_copy(tab_hbm.at[idxv.at[0]], rows.at[pl.ds(0, 112)], sem)
        cp1 = pltpu.async_copy(tab_hbm.at[idxv.at[1]], rows.at[pl.ds(112, 112)], sem)
        cp0.wait()
        cp1.wait()
        base = (b * L_ + i) * L_
        pltpu.sync_copy(rows.at[pl.ds(0, L_)], out_hbm.at[pl.ds(base, L_)])
        return carry

    lax.fori_loop(0, L_ // 2, unit, 0)


_sc_call = functools.partial(
    pl.kernel,
    mesh=plsc.VectorSubcoreMesh(core_axis_name="c", subcore_axis_name="s"),
    out_type=jax.ShapeDtypeStruct((B_ * L_ * L_, DIM_), jnp.float32),
    scratch_types=[
        pltpu.VMEM((LPAD_,), jnp.float32),
        pltpu.VMEM((LPAD_,), jnp.float32),
        pltpu.VMEM((2, 112), jnp.int32),
        pltpu.VMEM((LPAD_, DIM_), jnp.float32),
        pltpu.SemaphoreType.DMA,
    ],
)(_sc_body)


def kernel(gt_bboxes, x_emb, y_emb):
    bbT = jnp.transpose(
        jnp.pad(gt_bboxes, ((0, 0), (0, LPAD_ - L_), (0, 0))), (2, 0, 1)
    )
    ax, ay = _avg_call(bbT)
    tab = _tab_call(x_emb, y_emb).reshape(256 * 256, DIM_)
    out = _sc_call(tab, ax, ay)
    return out.reshape(B_, L_, L_, DIM_)


# speedup vs baseline: 6.0985x; 6.0985x over previous
import functools

import jax
import jax.numpy as jnp
from jax import lax
from jax.experimental import pallas as pl
from jax.experimental.pallas import tpu as pltpu
from jax.experimental.pallas import tpu_sc as plsc

MPE_ = 128
DIM_ = 64
B_ = 16
L_ = 200
LPAD_ = 224
NW_ = 32


def _avg_body(b_ref, ax_ref, ay_ref):
    ax_ref[...] = (b_ref[0] * float(MPE_) + b_ref[2] * float(MPE_)) * 0.5
    ay_ref[...] = (b_ref[1] * float(MPE_) + b_ref[3] * float(MPE_)) * 0.5


_avg_call = pl.pallas_call(
    _avg_body,
    out_shape=(
        jax.ShapeDtypeStruct((B_, LPAD_), jnp.float32),
        jax.ShapeDtypeStruct((B_, LPAD_), jnp.float32),
    ),
)


def _tab_body(x_ref, y_ref, o_ref):
    o_ref[...] = x_ref[...][:, None, :] + y_ref[...][None, :, :]


_tab_call = pl.pallas_call(
    _tab_body,
    grid=(8,),
    in_specs=[
        pl.BlockSpec((32, DIM_), lambda i: (i, 0)),
        pl.BlockSpec((256, DIM_), lambda i: (0, 0)),
    ],
    out_specs=pl.BlockSpec((32, 256, DIM_), lambda i: (i, 0, 0)),
    out_shape=jax.ShapeDtypeStruct((256, 256, DIM_), jnp.float32),
)


def _sc_body(tab_hbm, ax_hbm, ay_hbm, out_hbm, axv, ayv, idxv, rows, sem):
    cid = lax.axis_index("c")
    sid = lax.axis_index("s")
    b = sid
    i0 = cid * (L_ // 2)

    pltpu.sync_copy(ax_hbm.at[b], axv)
    pltpu.sync_copy(ay_hbm.at[b], ayv)

    def unit(i_rel, carry):
        i = i0 + i_rel
        axi = axv[pl.ds(i, 16)][0]
        ayi = ayv[pl.ds(i, 16)][0]
        for k in range(LPAD_ // 16):
            aj = axv[pl.ds(k * 16, 16)]
            dx = axi - aj + float(MPE_)
            ix = jnp.clip(dx, 0.0, float(2 * MPE_ - 1)).astype(jnp.int32)
            bj = ayv[pl.ds(k * 16, 16)]
            dy = ayi - bj + float(MPE_)
            iy = jnp.clip(dy, 0.0, float(2 * MPE_ - 1)).astype(jnp.int32)
            idx16 = ix * 256 + iy
            idxv[k // 7, pl.ds((k % 7) * 16, 16)] = idx16
        cp0 = pltpu.async_copy(tab_hbm.at[idxv.at[0]], rows.at[pl.ds(0, 112)], sem)
        cp1 = pltpu.async_copy(tab_hbm.at[idxv.at[1]], rows.at[pl.ds(112, 112)], sem)
        cp0.wait()
        cp1.wait()
        base = (b * L_ + i) * L_
        pltpu.sync_copy(rows.at[pl.ds(0, L_)], out_hbm.at[pl.ds(base, L_)])
        return carry

    lax.fori_loop(0, L_ // 2, unit, 0)


@functools.lru_cache(maxsize=1)
def _get_sc_call():
    return functools.partial(
        pl.kernel,
        mesh=plsc.VectorSubcoreMesh(core_axis_name="c", subcore_axis_name="s"),
        compiler_params=pltpu.CompilerParams(use_tc_tiling_on_sc=False),
        out_type=jax.ShapeDtypeStruct((B_ * L_ * L_, DIM_), jnp.float32),
        scratch_types=[
            pltpu.VMEM((LPAD_,), jnp.float32),
            pltpu.VMEM((LPAD_,), jnp.float32),
            pltpu.VMEM((2, 112), jnp.int32),
            pltpu.VMEM((LPAD_, DIM_), jnp.float32),
            pltpu.SemaphoreType.DMA,
        ],
    )(_sc_body)


def kernel(gt_bboxes, x_emb, y_emb):
    bbT = jnp.transpose(
        jnp.pad(gt_bboxes, ((0, 0), (0, LPAD_ - L_), (0, 0))), (2, 0, 1)
    )
    ax, ay = _avg_call(bbT)
    tab = _tab_call(x_emb, y_emb).reshape(256 * 256, DIM_)
    out = _get_sc_call()(tab, ax, ay)
    return out.reshape(B_, L_, L_, DIM_)


# speedup vs baseline: 6.2043x; 1.0174x over previous
import functools

import jax
import jax.numpy as jnp
from jax import lax
from jax.experimental import pallas as pl
from jax.experimental.pallas import tpu as pltpu
from jax.experimental.pallas import tpu_sc as plsc

MPE_ = 128
DIM_ = 64
B_ = 16
L_ = 200
LPAD_ = 224
NW_ = 32


def _avg_body(b_ref, ax_ref, ay_ref):
    ax_ref[...] = (b_ref[0] * float(MPE_) + b_ref[2] * float(MPE_)) * 0.5
    ay_ref[...] = (b_ref[1] * float(MPE_) + b_ref[3] * float(MPE_)) * 0.5


_avg_call = pl.pallas_call(
    _avg_body,
    out_shape=(
        jax.ShapeDtypeStruct((B_, LPAD_), jnp.float32),
        jax.ShapeDtypeStruct((B_, LPAD_), jnp.float32),
    ),
)


def _tab_body(x_ref, y_ref, o_ref):
    o_ref[...] = x_ref[...][:, None, :] + y_ref[...][None, :, :]


_tab_call = pl.pallas_call(
    _tab_body,
    grid=(8,),
    in_specs=[
        pl.BlockSpec((32, DIM_), lambda i: (i, 0)),
        pl.BlockSpec((256, DIM_), lambda i: (0, 0)),
    ],
    out_specs=pl.BlockSpec((32, 256, DIM_), lambda i: (i, 0, 0)),
    out_shape=jax.ShapeDtypeStruct((256, 256, DIM_), jnp.float32),
)


NUNIT_ = L_ // 2
GSZ_ = 4
GROUPS_ = NUNIT_ // GSZ_
GROWS_ = GSZ_ * L_
ISTR_ = 208
NIDX_ = NUNIT_ * ISTR_


def _sc_body(tab_hbm, ax_hbm, ay_hbm, out_hbm, axv, ayv, idxv, rows,
             sem_g, sem_o):
    cid = lax.axis_index("c")
    sid = lax.axis_index("s")
    b = sid
    i0 = cid * NUNIT_
    wbase = (b * L_ + i0) * L_

    pltpu.sync_copy(ax_hbm.at[b], axv)
    pltpu.sync_copy(ay_hbm.at[b], ayv)

    def unit(u, carry):
        i = i0 + u
        axi = axv[pl.ds(i, 16)][0]
        ayi = ayv[pl.ds(i, 16)][0]
        for k in range(13):
            aj = axv[pl.ds(k * 16, 16)]
            dx = axi - aj + float(MPE_)
            ix = jnp.clip(dx, 0.0, float(2 * MPE_ - 1)).astype(jnp.int32)
            bj = ayv[pl.ds(k * 16, 16)]
            dy = ayi - bj + float(MPE_)
            iy = jnp.clip(dy, 0.0, float(2 * MPE_ - 1)).astype(jnp.int32)
            idxv[pl.ds(u * ISTR_ + k * 16, 16)] = ix * 256 + iy
        return carry

    lax.fori_loop(0, NUNIT_, unit, 0)

    def group(g, carry):
        off = (g % 2) * GROWS_

        @pl.when(g >= 2)
        def _():
            pltpu.make_async_copy(
                out_hbm.at[pl.ds(0, GROWS_)], rows.at[pl.ds(off, GROWS_)],
                sem_o).wait()

        cps = []
        for s in range(GSZ_):
            jb = (g * GSZ_ + s) * ISTR_
            dst = off + s * L_
            cps.append(pltpu.async_copy(
                tab_hbm.at[idxv.at[pl.ds(jb, 104)]],
                rows.at[pl.ds(dst, 104)], sem_g))
            cps.append(pltpu.async_copy(
                tab_hbm.at[idxv.at[pl.ds(jb + 104, 96)]],
                rows.at[pl.ds(dst + 104, 96)], sem_g))
        for cp in cps:
            cp.wait()
        pltpu.async_copy(rows.at[pl.ds(off, GROWS_)],
                         out_hbm.at[pl.ds(wbase + g * GROWS_, GROWS_)], sem_o)
        return carry

    lax.fori_loop(0, GROUPS_, group, 0)
    for _ in range(2):
        pltpu.make_async_copy(out_hbm.at[pl.ds(0, GROWS_)],
                              rows.at[pl.ds(0, GROWS_)], sem_o).wait()


@functools.lru_cache(maxsize=1)
def _get_sc_call():
    return functools.partial(
        pl.kernel,
        mesh=plsc.VectorSubcoreMesh(core_axis_name="c", subcore_axis_name="s"),
        compiler_params=pltpu.CompilerParams(use_tc_tiling_on_sc=False),
        out_type=jax.ShapeDtypeStruct((B_ * L_ * L_, DIM_), jnp.float32),
        scratch_types=[
            pltpu.VMEM((LPAD_,), jnp.float32),
            pltpu.VMEM((LPAD_,), jnp.float32),
            pltpu.VMEM((NIDX_,), jnp.int32),
            pltpu.VMEM((2 * GROWS_, DIM_), jnp.float32),
            pltpu.SemaphoreType.DMA,
            pltpu.SemaphoreType.DMA,
        ],
    )(_sc_body)


def kernel(gt_bboxes, x_emb, y_emb):
    bbT = jnp.transpose(
        jnp.pad(gt_bboxes, ((0, 0), (0, LPAD_ - L_), (0, 0))), (2, 0, 1)
    )
    ax, ay = _avg_call(bbT)
    tab = _tab_call(x_emb, y_emb).reshape(256 * 256, DIM_)
    out = _get_sc_call()(tab, ax, ay)
    return out.reshape(B_, L_, L_, DIM_)


# speedup vs baseline: 6.4163x; 1.0342x over previous
import functools

import jax
import jax.numpy as jnp
from jax import lax
from jax.experimental import pallas as pl
from jax.experimental.pallas import tpu as pltpu
from jax.experimental.pallas import tpu_sc as plsc

MPE_ = 128
DIM_ = 64
B_ = 16
L_ = 200
LPAD_ = 224
NW_ = 32
ROWS_PER_W_ = 256 // NW_
TROW_ = 256 * DIM_


def _build_body(x_hbm, y_hbm, bb_hbm, tab_hbm, ax_hbm, ay_hbm,
                xv, yv, bbv, axb, ayb, tb, sem_t):
    cid = lax.axis_index("c")
    sid = lax.axis_index("s")
    w = sid * 2 + cid

    pltpu.sync_copy(y_hbm, yv)
    pltpu.sync_copy(x_hbm.at[pl.ds(w * (ROWS_PER_W_ * DIM_), ROWS_PER_W_ * DIM_)], xv)

    @pl.when(w < B_)
    def _():
        for c in range(4):
            pltpu.sync_copy(bb_hbm.at[c * B_ + w], bbv.at[c])
        for k in range(LPAD_ // 16):
            sl = pl.ds(k * 16, 16)
            axb[sl] = (bbv[0, sl] * float(MPE_) + bbv[2, sl] * float(MPE_)) * 0.5
            ayb[sl] = (bbv[1, sl] * float(MPE_) + bbv[3, sl] * float(MPE_)) * 0.5
        pltpu.sync_copy(axb, ax_hbm.at[w])
        pltpu.sync_copy(ayb, ay_hbm.at[w])

    for r in range(ROWS_PER_W_):
        slot = r % 2
        if r >= 2:
            pltpu.make_async_copy(tab_hbm.at[pl.ds(0, TROW_)],
                                  tb.at[pl.ds(slot * TROW_, TROW_)],
                                  sem_t).wait()
        xr = [xv[pl.ds(r * DIM_ + c * 16, 16)] for c in range(4)]

        def iyb(iy, carry, slot=slot, xr=xr):
            o = iy * DIM_
            for c in range(4):
                tb[pl.ds(slot * TROW_ + o + c * 16, 16)] = (
                    yv[pl.ds(o + c * 16, 16)] + xr[c])
            return carry

        lax.fori_loop(0, 256, iyb, 0)
        pltpu.async_copy(
            tb.at[pl.ds(slot * TROW_, TROW_)],
            tab_hbm.at[pl.ds((w * ROWS_PER_W_ + r) * TROW_, TROW_)], sem_t)

    for s in range(2):
        pltpu.make_async_copy(tab_hbm.at[pl.ds(0, TROW_)],
                              tb.at[pl.ds(s * TROW_, TROW_)], sem_t).wait()


NUNIT_ = L_ // 2
GSZ_ = 4
GROUPS_ = NUNIT_ // GSZ_
GROWS_ = GSZ_ * L_
ISTR_ = 208
NIDX_ = NUNIT_ * ISTR_


def _sc_body(tab_hbm, ax_hbm, ay_hbm, out_hbm, axv, ayv, idxv, rows,
             sem_g, sem_o):
    cid = lax.axis_index("c")
    sid = lax.axis_index("s")
    b = sid
    i0 = cid * NUNIT_
    wbase = (b * L_ + i0) * L_

    pltpu.sync_copy(ax_hbm.at[b], axv)
    pltpu.sync_copy(ay_hbm.at[b], ayv)

    def unit(u, carry):
        i = i0 + u
        axi = axv[pl.ds(i, 16)][0]
        ayi = ayv[pl.ds(i, 16)][0]
        for k in range(13):
            aj = axv[pl.ds(k * 16, 16)]
            dx = axi - aj + float(MPE_)
            ix = jnp.clip(dx, 0.0, float(2 * MPE_ - 1)).astype(jnp.int32)
            bj = ayv[pl.ds(k * 16, 16)]
            dy = ayi - bj + float(MPE_)
            iy = jnp.clip(dy, 0.0, float(2 * MPE_ - 1)).astype(jnp.int32)
            idxv[pl.ds(u * ISTR_ + k * 16, 16)] = ix * 256 + iy
        return carry

    lax.fori_loop(0, NUNIT_, unit, 0)

    def group(g, carry):
        off = (g % 2) * GROWS_

        @pl.when(g >= 2)
        def _():
            pltpu.make_async_copy(
                out_hbm.at[pl.ds(0, GROWS_)], rows.at[pl.ds(off, GROWS_)],
                sem_o).wait()

        cps = []
        for s in range(GSZ_):
            jb = (g * GSZ_ + s) * ISTR_
            dst = off + s * L_
            cps.append(pltpu.async_copy(
                tab_hbm.at[idxv.at[pl.ds(jb, 104)]],
                rows.at[pl.ds(dst, 104)], sem_g))
            cps.append(pltpu.async_copy(
                tab_hbm.at[idxv.at[pl.ds(jb + 104, 96)]],
                rows.at[pl.ds(dst + 104, 96)], sem_g))
        for cp in cps:
            cp.wait()
        pltpu.async_copy(rows.at[pl.ds(off, GROWS_)],
                         out_hbm.at[pl.ds(wbase + g * GROWS_, GROWS_)], sem_o)
        return carry

    lax.fori_loop(0, GROUPS_, group, 0)
    for _ in range(2):
        pltpu.make_async_copy(out_hbm.at[pl.ds(0, GROWS_)],
                              rows.at[pl.ds(0, GROWS_)], sem_o).wait()


@functools.lru_cache(maxsize=1)
def _get_calls():
    mesh = plsc.VectorSubcoreMesh(core_axis_name="c", subcore_axis_name="s")
    params = pltpu.CompilerParams(use_tc_tiling_on_sc=False)
    build = functools.partial(
        pl.kernel,
        mesh=mesh,
        compiler_params=params,
        out_type=(
            jax.ShapeDtypeStruct((256 * 256 * DIM_,), jnp.float32),
            jax.ShapeDtypeStruct((B_, LPAD_), jnp.float32),
            jax.ShapeDtypeStruct((B_, LPAD_), jnp.float32),
        ),
        scratch_types=[
            pltpu.VMEM((ROWS_PER_W_ * DIM_,), jnp.float32),
            pltpu.VMEM((256 * DIM_,), jnp.float32),
            pltpu.VMEM((4, LPAD_), jnp.float32),
            pltpu.VMEM((LPAD_,), jnp.float32),
            pltpu.VMEM((LPAD_,), jnp.float32),
            pltpu.VMEM((2 * TROW_,), jnp.float32),
            pltpu.SemaphoreType.DMA,
        ],
    )(_build_body)
    gather = functools.partial(
        pl.kernel,
        mesh=mesh,
        compiler_params=params,
        out_type=jax.ShapeDtypeStruct((B_ * L_ * L_, DIM_), jnp.float32),
        scratch_types=[
            pltpu.VMEM((LPAD_,), jnp.float32),
            pltpu.VMEM((LPAD_,), jnp.float32),
            pltpu.VMEM((NIDX_,), jnp.int32),
            pltpu.VMEM((2 * GROWS_, DIM_), jnp.float32),
            pltpu.SemaphoreType.DMA,
            pltpu.SemaphoreType.DMA,
        ],
    )(_sc_body)
    return build, gather


def kernel(gt_bboxes, x_emb, y_emb):
    bbT = jnp.transpose(
        jnp.pad(gt_bboxes, ((0, 0), (0, LPAD_ - L_), (0, 0))), (2, 0, 1)
    ).reshape(4 * B_, LPAD_)
    build, gather = _get_calls()
    tab, ax, ay = build(x_emb.reshape(-1), y_emb.reshape(-1), bbT)
    out = gather(tab.reshape(256 * 256, DIM_), ax, ay)
    return out.reshape(B_, L_, L_, DIM_)


# speedup vs baseline: 6.4205x; 1.0006x over previous
import functools

import jax
import jax.numpy as jnp
from jax import lax
from jax.experimental import pallas as pl
from jax.experimental.pallas import tpu as pltpu
from jax.experimental.pallas import tpu_sc as plsc

MPE_ = 128
DIM_ = 64
B_ = 16
L_ = 200
LPAD_ = 224
NW_ = 32
ROWS_PER_W_ = 256 // NW_
TROW_ = 256 * DIM_


def _build_body(x_hbm, y_hbm, bb_hbm, tab_hbm, ax_hbm, ay_hbm,
                xv, yv, bbv, axb, ayb, tb, sem_t):
    cid = lax.axis_index("c")
    sid = lax.axis_index("s")
    w = sid * 2 + cid

    pltpu.sync_copy(y_hbm, yv)
    pltpu.sync_copy(x_hbm.at[pl.ds(w * (ROWS_PER_W_ * DIM_), ROWS_PER_W_ * DIM_)], xv)

    @pl.when(w < B_)
    def _():
        for c in range(4):
            pltpu.sync_copy(bb_hbm.at[c * B_ + w], bbv.at[c])
        for k in range(LPAD_ // 16):
            sl = pl.ds(k * 16, 16)
            axb[sl] = (bbv[0, sl] * float(MPE_) + bbv[2, sl] * float(MPE_)) * 0.5
            ayb[sl] = (bbv[1, sl] * float(MPE_) + bbv[3, sl] * float(MPE_)) * 0.5
        pltpu.sync_copy(axb, ax_hbm.at[w])
        pltpu.sync_copy(ayb, ay_hbm.at[w])

    for r in range(ROWS_PER_W_):
        slot = r % 2
        if r >= 2:
            pltpu.make_async_copy(tab_hbm.at[pl.ds(0, 256)],
                                  tb.at[pl.ds(slot * 256, 256)],
                                  sem_t).wait()
        xr = [xv[pl.ds(r * DIM_ + c * 16, 16)] for c in range(4)]

        def iyb(iy, carry, slot=slot, xr=xr):
            row = slot * 256 + iy
            for c in range(4):
                tb[row, pl.ds(c * 16, 16)] = (
                    yv[pl.ds(iy * DIM_ + c * 16, 16)] + xr[c])
            return carry

        lax.fori_loop(0, 256, iyb, 0)
        pltpu.async_copy(
            tb.at[pl.ds(slot * 256, 256)],
            tab_hbm.at[pl.ds((w * ROWS_PER_W_ + r) * 256, 256)], sem_t)

    for s in range(2):
        pltpu.make_async_copy(tab_hbm.at[pl.ds(0, 256)],
                              tb.at[pl.ds(s * 256, 256)], sem_t).wait()


NUNIT_ = L_ // 2
GSZ_ = 4
GROUPS_ = NUNIT_ // GSZ_
GROWS_ = GSZ_ * L_
ISTR_ = 208
NIDX_ = NUNIT_ * ISTR_


def _sc_body(tab_hbm, ax_hbm, ay_hbm, out_hbm, axv, ayv, idxv, rows,
             sem_g, sem_o):
    cid = lax.axis_index("c")
    sid = lax.axis_index("s")
    b = sid
    i0 = cid * NUNIT_
    wbase = (b * L_ + i0) * L_

    pltpu.sync_copy(ax_hbm.at[b], axv)
    pltpu.sync_copy(ay_hbm.at[b], ayv)

    def unit(u, carry):
        i = i0 + u
        axi = axv[pl.ds(i, 16)][0]
        ayi = ayv[pl.ds(i, 16)][0]
        for k in range(13):
            aj = axv[pl.ds(k * 16, 16)]
            dx = axi - aj + float(MPE_)
            ix = jnp.clip(dx, 0.0, float(2 * MPE_ - 1)).astype(jnp.int32)
            bj = ayv[pl.ds(k * 16, 16)]
            dy = ayi - bj + float(MPE_)
            iy = jnp.clip(dy, 0.0, float(2 * MPE_ - 1)).astype(jnp.int32)
            idxv[pl.ds(u * ISTR_ + k * 16, 16)] = ix * 256 + iy
        return carry

    lax.fori_loop(0, NUNIT_, unit, 0)

    def group(g, carry):
        off = (g % 2) * GROWS_

        @pl.when(g >= 2)
        def _():
            pltpu.make_async_copy(
                out_hbm.at[pl.ds(0, GROWS_)], rows.at[pl.ds(off, GROWS_)],
                sem_o).wait()

        cps = []
        for s in range(GSZ_):
            jb = (g * GSZ_ + s) * ISTR_
            dst = off + s * L_
            cps.append(pltpu.async_copy(
                tab_hbm.at[idxv.at[pl.ds(jb, 104)]],
                rows.at[pl.ds(dst, 104)], sem_g))
            cps.append(pltpu.async_copy(
                tab_hbm.at[idxv.at[pl.ds(jb + 104, 96)]],
                rows.at[pl.ds(dst + 104, 96)], sem_g))
        for cp in cps:
            cp.wait()
        pltpu.async_copy(rows.at[pl.ds(off, GROWS_)],
                         out_hbm.at[pl.ds(wbase + g * GROWS_, GROWS_)], sem_o)
        return carry

    lax.fori_loop(0, GROUPS_, group, 0)
    for _ in range(2):
        pltpu.make_async_copy(out_hbm.at[pl.ds(0, GROWS_)],
                              rows.at[pl.ds(0, GROWS_)], sem_o).wait()


@functools.lru_cache(maxsize=1)
def _get_calls():
    mesh = plsc.VectorSubcoreMesh(core_axis_name="c", subcore_axis_name="s")
    params = pltpu.CompilerParams(use_tc_tiling_on_sc=False)
    build = functools.partial(
        pl.kernel,
        mesh=mesh,
        compiler_params=params,
        out_type=(
            jax.ShapeDtypeStruct((256 * 256, DIM_), jnp.float32),
            jax.ShapeDtypeStruct((B_, LPAD_), jnp.float32),
            jax.ShapeDtypeStruct((B_, LPAD_), jnp.float32),
        ),
        scratch_types=[
            pltpu.VMEM((ROWS_PER_W_ * DIM_,), jnp.float32),
            pltpu.VMEM((256 * DIM_,), jnp.float32),
            pltpu.VMEM((4, LPAD_), jnp.float32),
            pltpu.VMEM((LPAD_,), jnp.float32),
            pltpu.VMEM((LPAD_,), jnp.float32),
            pltpu.VMEM((2 * 256, DIM_), jnp.float32),
            pltpu.SemaphoreType.DMA,
        ],
    )(_build_body)
    gather = functools.partial(
        pl.kernel,
        mesh=mesh,
        compiler_params=params,
        out_type=jax.ShapeDtypeStruct((B_ * L_ * L_, DIM_), jnp.float32),
        scratch_types=[
            pltpu.VMEM((LPAD_,), jnp.float32),
            pltpu.VMEM((LPAD_,), jnp.float32),
            pltpu.VMEM((NIDX_,), jnp.int32),
            pltpu.VMEM((2 * GROWS_, DIM_), jnp.float32),
            pltpu.SemaphoreType.DMA,
            pltpu.SemaphoreType.DMA,
        ],
    )(_sc_body)
    return build, gather


def kernel(gt_bboxes, x_emb, y_emb):
    bbT = jnp.transpose(
        jnp.pad(gt_bboxes, ((0, 0), (0, LPAD_ - L_), (0, 0))), (2, 0, 1)
    ).reshape(4 * B_, LPAD_)
    build, gather = _get_calls()
    tab, ax, ay = build(x_emb.reshape(-1), y_emb.reshape(-1), bbT)
    out = gather(tab, ax, ay)
    return out.reshape(B_, L_, L_, DIM_)
